# baseline (speedup 1.0000x reference)
import functools

import jax
import jax.numpy as jnp
from jax import lax
from jax.experimental import pallas as pl
from jax.experimental.pallas import tpu as pltpu

D_MODEL = 2048
N_EXP = 16
BLK_T = 1024


def _router_kernel(x_ref, w_ref, b_ref, out_ref, aux_ref,
                   cnt_ref, ps_ref, *, num_blocks, num_tokens):
    i = pl.program_id(0)

    @pl.when(i == 0)
    def _init():
        cnt_ref[...] = jnp.zeros_like(cnt_ref)
        ps_ref[...] = jnp.zeros_like(ps_ref)

    logits = lax.dot_general(
        w_ref[...], x_ref[...], (((1,), (1,)), ((), ())),
        preferred_element_type=jnp.float32) + b_ref[...]
    eidx = lax.broadcasted_iota(jnp.int32, logits.shape, 0)
    m = jnp.max(logits, axis=0, keepdims=True)
    idx1 = jnp.min(jnp.where(logits == m, eidx, N_EXP),
                   axis=0, keepdims=True)
    hit1 = eidx == idx1
    masked = jnp.where(hit1, -jnp.inf, logits)
    max2 = jnp.max(masked, axis=0, keepdims=True)
    idx2 = jnp.min(jnp.where(masked == max2, eidx, N_EXP),
                   axis=0, keepdims=True)
    hit2 = eidx == idx2

    e = jnp.exp(logits - m)
    r = 1.0 / jnp.sum(e, axis=0, keepdims=True)
    probs = e * r
    score2 = jnp.exp(max2 - m) * r

    out_ref[...] = jnp.concatenate(
        [idx1.astype(jnp.float32), idx2.astype(jnp.float32), r, score2],
        axis=0)

    cnt_ref[...] += jnp.sum(
        hit1.astype(jnp.float32) + hit2.astype(jnp.float32),
        axis=1, keepdims=True)
    ps_ref[...] += jnp.sum(probs, axis=1, keepdims=True)

    @pl.when(i == num_blocks - 1)
    def _fin():
        inv = 1.0 / num_tokens
        aux_ref[...] = N_EXP * jnp.sum(
            (cnt_ref[...] * inv) * (ps_ref[...] * inv),
            axis=(0, 1), keepdims=True)


@jax.jit
def kernel(x, W, b):
    B, S, D = x.shape
    num_tokens = B * S
    num_blocks = num_tokens // BLK_T
    xf = x.reshape(num_tokens, D)
    b2 = b.reshape(N_EXP, 1)

    outT, aux = pl.pallas_call(
        functools.partial(_router_kernel, num_blocks=num_blocks,
                          num_tokens=num_tokens),
        grid=(num_blocks,),
        in_specs=[
            pl.BlockSpec((BLK_T, D), lambda i: (i, 0)),
            pl.BlockSpec((N_EXP, D), lambda i: (0, 0)),
            pl.BlockSpec((N_EXP, 1), lambda i: (0, 0)),
        ],
        out_specs=[
            pl.BlockSpec((4, BLK_T), lambda i: (0, i)),
            pl.BlockSpec((1, 1), lambda i: (0, 0)),
        ],
        out_shape=[
            jax.ShapeDtypeStruct((4, num_tokens), jnp.float32),
            jax.ShapeDtypeStruct((1, 1), jnp.float32),
        ],
        scratch_shapes=[
            pltpu.VMEM((N_EXP, 1), jnp.float32),
            pltpu.VMEM((N_EXP, 1), jnp.float32),
        ],
        compiler_params=pltpu.CompilerParams(
            dimension_semantics=("arbitrary",),
        ),
    )(xf, W, b2)

    o = outT.T.reshape(B, S, 4)
    idx = o[..., :2].astype(jnp.int32)
    score = o[..., 2:]
    return (idx, score, aux[0, 0])


# speedup vs baseline: 1.0427x; 1.0427x over previous
import functools

import jax
import jax.numpy as jnp
from jax import lax
from jax.experimental import pallas as pl
from jax.experimental.pallas import tpu as pltpu

D_MODEL = 2048
N_EXP = 16
BLK_T = 1024


def _router_kernel(x_ref, w_ref, b_ref, idx_ref, score_ref, aux_ref,
                   cnt_ref, ps_ref, *, num_blocks, num_tokens):
    i = pl.program_id(0)

    @pl.when(i == 0)
    def _init():
        cnt_ref[...] = jnp.zeros_like(cnt_ref)
        ps_ref[...] = jnp.zeros_like(ps_ref)

    logits = lax.dot_general(
        w_ref[...], x_ref[...], (((1,), (1,)), ((), ())),
        preferred_element_type=jnp.float32) + b_ref[...]
    eidx = lax.broadcasted_iota(jnp.int32, logits.shape, 0)
    m = jnp.max(logits, axis=0, keepdims=True)
    idx1 = jnp.min(jnp.where(logits == m, eidx, N_EXP),
                   axis=0, keepdims=True)
    hit1 = eidx == idx1
    masked = jnp.where(hit1, -jnp.inf, logits)
    max2 = jnp.max(masked, axis=0, keepdims=True)
    idx2 = jnp.min(jnp.where(masked == max2, eidx, N_EXP),
                   axis=0, keepdims=True)
    hit2 = eidx == idx2

    e = jnp.exp(logits - m)
    r = 1.0 / jnp.sum(e, axis=0, keepdims=True)
    probs = e * r
    score2 = jnp.exp(max2 - m) * r

    idx_ref[...] = jnp.concatenate([idx1, idx2], axis=0)
    score_ref[...] = jnp.concatenate([r, score2], axis=0)

    cnt_ref[...] += jnp.sum(
        hit1.astype(jnp.float32) + hit2.astype(jnp.float32),
        axis=1, keepdims=True)
    ps_ref[...] += jnp.sum(probs, axis=1, keepdims=True)

    @pl.when(i == num_blocks - 1)
    def _fin():
        inv = 1.0 / num_tokens
        aux_ref[...] = N_EXP * jnp.sum(
            (cnt_ref[...] * inv) * (ps_ref[...] * inv),
            axis=(0, 1), keepdims=True)


@jax.jit
def kernel(x, W, b):
    B, S, D = x.shape
    num_tokens = B * S
    num_blocks = num_tokens // BLK_T
    xf = x.reshape(num_tokens, D)
    b2 = b.reshape(N_EXP, 1)

    idxT, scoreT, aux = pl.pallas_call(
        functools.partial(_router_kernel, num_blocks=num_blocks,
                          num_tokens=num_tokens),
        grid=(num_blocks,),
        in_specs=[
            pl.BlockSpec((BLK_T, D), lambda i: (i, 0)),
            pl.BlockSpec((N_EXP, D), lambda i: (0, 0)),
            pl.BlockSpec((N_EXP, 1), lambda i: (0, 0)),
        ],
        out_specs=[
            pl.BlockSpec((2, BLK_T), lambda i: (0, i)),
            pl.BlockSpec((2, BLK_T), lambda i: (0, i)),
            pl.BlockSpec((1, 1), lambda i: (0, 0)),
        ],
        out_shape=[
            jax.ShapeDtypeStruct((2, num_tokens), jnp.int32),
            jax.ShapeDtypeStruct((2, num_tokens), jnp.float32),
            jax.ShapeDtypeStruct((1, 1), jnp.float32),
        ],
        scratch_shapes=[
            pltpu.VMEM((N_EXP, 1), jnp.float32),
            pltpu.VMEM((N_EXP, 1), jnp.float32),
        ],
        compiler_params=pltpu.CompilerParams(
            dimension_semantics=("arbitrary",),
        ),
    )(xf, W, b2)

    idx = idxT.T.reshape(B, S, 2)
    score = scoreT.T.reshape(B, S, 2)
    return (idx, score, aux[0, 0])
